# baseline (speedup 1.0000x reference)
import functools
import math

import jax
import jax.numpy as jnp
from jax import lax
from jax.experimental import pallas as pl
from jax.experimental.pallas import tpu as pltpu
from jax.experimental.pallas import tpu_sc as plsc

EMBED_DIM = 64
EMBED_SCALE = math.sqrt(EMBED_DIM)
VOCAB = 1000000
NLINES = VOCAB // 2
NFULL = VOCAB // 128
NRAG = VOCAB - NFULL * 128

_INFO = None


def _info():
    global _INFO
    if _INFO is None:
        _INFO = plsc.get_sparse_core_info()
    return _INFO


def _iota16():
    return lax.iota(jnp.int32, 16)


def _sc_build_lines(wt, wt_tail):
    info = _info()
    nw = info.num_cores * info.num_subcores
    per_w = -(-NFULL // nw)

    mesh = plsc.VectorSubcoreMesh(core_axis_name="c", subcore_axis_name="s")

    @functools.partial(
        pl.kernel,
        mesh=mesh,
        out_type=jax.ShapeDtypeStruct((NLINES, 128), jnp.float32),
        scratch_types=[
            *[pltpu.VMEM((64, 128), jnp.float32) for _ in range(2)],
            *[pltpu.VMEM((64, 128), jnp.float32) for _ in range(2)],
            pltpu.VMEM((64, 64), jnp.float32),
            *[pltpu.SemaphoreType.DMA for _ in range(4)],
        ],
        compiler_params=pltpu.CompilerParams(needs_layout_passes=False),
    )
    def k(wt_hbm, tail_hbm, lines_hbm, sa0, sa1, lb0, lb1, tslab, g0, g1, s0, s1):
        slab = (sa0, sa1)
        lbuf = (lb0, lb1)
        gsem = (g0, g1)
        ssem = (s0, s1)
        wid = lax.axis_index("s") * info.num_cores + lax.axis_index("c")

        def fire_slab(b, blk):
            pltpu.async_copy(
                wt_hbm.at[:, pl.ds(blk * 128, 128)], slab[b], gsem[b]
            )

        def transpose_block(b):
            @plsc.parallel_loop(0, 64, unroll=2)
            def _(j):
                c0 = jnp.broadcast_to(2 * j, (16,)).astype(jnp.int32)
                c1 = c0 + 1
                for g in range(8):
                    rows = _iota16() + (16 * g) % 64
                    col = c0 if g < 4 else c1
                    v = plsc.load_gather(slab[b], [rows, col]) * EMBED_SCALE
                    lbuf[b][j, pl.ds(16 * g, 16)] = v

        for b in range(2):
            blk = wid + nw * b
            @pl.when(blk < NFULL)
            def _():
                fire_slab(b, blk)

        def body(i, carry):
            for b in range(2):
                blk = wid + nw * (i * 2 + b)

                @pl.when(blk < NFULL)
                def _():
                    pltpu.make_async_copy(
                        wt_hbm.at[:, pl.ds(0, 128)], slab[b], gsem[b]
                    ).wait()

                    @pl.when(i > 0)
                    def _():
                        pltpu.make_async_copy(
                            lbuf[b], lines_hbm.at[pl.ds(0, 64), :], ssem[b]
                        ).wait()

                    transpose_block(b)
                    pltpu.async_copy(
                        lbuf[b], lines_hbm.at[pl.ds(blk * 64, 64), :], ssem[b]
                    )
                    nxt = blk + nw * 2
                    @pl.when(nxt < NFULL)
                    def _():
                        fire_slab(b, nxt)
            return carry

        lax.fori_loop(0, per_w // 2 + 1, body, 0)

        for b in range(2):
            blk = wid + nw * b
            @pl.when(blk < NFULL)
            def _():
                pltpu.make_async_copy(
                    lbuf[b], lines_hbm.at[pl.ds(0, 64), :], ssem[b]
                ).wait()

        @pl.when(wid == 0)
        def _():
            pltpu.sync_copy(tail_hbm, tslab)

            @plsc.parallel_loop(0, 32, unroll=2)
            def _(j):
                c0 = jnp.broadcast_to(2 * j, (16,)).astype(jnp.int32)
                c1 = c0 + 1
                for g in range(8):
                    rows = _iota16() + (16 * g) % 64
                    col = c0 if g < 4 else c1
                    v = plsc.load_gather(tslab, [rows, col]) * EMBED_SCALE
                    lbuf[0][j, pl.ds(16 * g, 16)] = v

            pltpu.sync_copy(
                lbuf[0].at[pl.ds(0, 32), :],
                lines_hbm.at[pl.ds(NFULL * 64, 32), :],
            )

    return k(wt, wt_tail)


def _sc_gather(idx_p, lines, S, N):
    info = _info()
    nw = info.num_cores * info.num_subcores
    C = 256
    n_chunks = (S * N) // C
    per_w = n_chunks // nw
    blocks_per_s = N // C
    assert per_w * nw == n_chunks and blocks_per_s * C == N
    blk_bits = blocks_per_s.bit_length() - 1

    mesh = plsc.VectorSubcoreMesh(core_axis_name="c", subcore_axis_name="s")

    @functools.partial(
        pl.kernel,
        mesh=mesh,
        out_type=jax.ShapeDtypeStruct((S, EMBED_DIM, N), jnp.float32),
        scratch_types=[
            *[pltpu.VMEM((C,), jnp.int32) for _ in range(2)],
            *[pltpu.VMEM((C,), jnp.int32) for _ in range(2)],
            *[pltpu.VMEM((C,), jnp.int32) for _ in range(2)],
            *[pltpu.VMEM((C, 128), jnp.float32) for _ in range(2)],
            *[pltpu.VMEM((EMBED_DIM, 128), jnp.float32) for _ in range(4)],
            *[pltpu.SemaphoreType.DMA for _ in range(4)],
        ],
        compiler_params=pltpu.CompilerParams(needs_layout_passes=False),
    )
    def k(idx_hbm, lines_hbm, out_hbm,
          p0_, p1_, l0_, l1_, h0_, h1_, lb0, lb1, t0, t1, t2, t3,
          g0, g1, s0, s1):
        pidx = (p0_, p1_)
        lidx = (l0_, l1_)
        h64 = (h0_, h1_)
        lbuf = (lb0, lb1)
        tbuf = ((t0, t1), (t2, t3))
        gsem = (g0, g1)
        ssem = (s0, s1)
        wid = lax.axis_index("s") * info.num_cores + lax.axis_index("c")
        base = wid * per_w

        def fire_gather(b, cid):
            pltpu.sync_copy(idx_hbm.at[pl.ds(cid * C, C)], pidx[b])

            @plsc.parallel_loop(0, C // 16, unroll=4)
            def _(r):
                t = pidx[b][pl.ds(r * 16, 16)]
                lidx[b][pl.ds(r * 16, 16)] = lax.shift_right_logical(t, 1)
                h64[b][pl.ds(r * 16, 16)] = lax.shift_left(t & 1, 6)

            pltpu.async_copy(lines_hbm.at[lidx[b]], lbuf[b], gsem[b])

        for b in range(2):
            fire_gather(b, base + b)

        def body(g, carry):
            for b in range(2):
                cid = base + g * 2 + b
                pltpu.make_async_copy(
                    lines_hbm.at[lidx[b]], lbuf[b], gsem[b]
                ).wait()

                @pl.when(g > 0)
                def _():
                    for nb in range(2):
                        pltpu.make_async_copy(
                            tbuf[b][nb], out_hbm.at[0, :, pl.ds(0, 128)],
                            ssem[b],
                        ).wait()

                s_pos = cid >> blk_bits
                n0 = (cid & (blocks_per_s - 1)) * C
                for nb in range(2):
                    @plsc.parallel_loop(0, 8, unroll=1)
                    def _(gg):
                        rows = _iota16() + gg * 16 + nb * 128
                        hv = h64[b][pl.ds(gg * 16 + nb * 128, 16)]
                        for d in range(EMBED_DIM):
                            col = hv + d
                            v = plsc.load_gather(lbuf[b], [rows, col])
                            tbuf[b][nb][d, pl.ds(gg * 16, 16)] = v

                for nb in range(2):
                    pltpu.async_copy(
                        tbuf[b][nb],
                        out_hbm.at[s_pos, :, pl.ds(n0 + nb * 128, 128)],
                        ssem[b],
                    )

                @pl.when(g + 1 < per_w // 2)
                def _():
                    fire_gather(b, cid + 2)
            return carry

        lax.fori_loop(0, per_w // 2, body, 0)

        for b in range(2):
            for nb in range(2):
                pltpu.make_async_copy(
                    tbuf[b][nb], out_hbm.at[0, :, pl.ds(0, 128)], ssem[b]
                ).wait()

    return k(idx_p, lines)


@jax.jit
def _run(tokens, weight):
    n, s = tokens.shape
    wt = weight.T
    wt_tail = lax.slice(wt, (0, NFULL * 128), (EMBED_DIM, VOCAB))
    lines = _sc_build_lines(wt, wt_tail)
    idx_p = tokens.T.reshape(-1)
    out_phys = _sc_gather(idx_p, lines, s, n)
    return jnp.transpose(out_phys, (2, 0, 1))


def kernel(tokens, weight):
    return _run(tokens, weight)


# speedup vs baseline: 1.9572x; 1.9572x over previous
import functools
import math

import jax
import jax.numpy as jnp
from jax import lax
from jax.experimental import pallas as pl
from jax.experimental.pallas import tpu as pltpu
from jax.experimental.pallas import tpu_sc as plsc

EMBED_DIM = 64
EMBED_SCALE = math.sqrt(EMBED_DIM)
VOCAB = 1000000
NLINES = VOCAB // 2
NFULL = VOCAB // 128
NRAG = VOCAB - NFULL * 128

_INFO = None


def _info():
    global _INFO
    if _INFO is None:
        _INFO = plsc.get_sparse_core_info()
    return _INFO


def _iota16():
    return lax.iota(jnp.int32, 16)


def _transpose_slab(slab, lbuf, n_j_blocks):
    iota = _iota16()

    @plsc.parallel_loop(0, 16, unroll=1)
    def _(k):
        dvec = (iota + k) & 15
        for jb in range(n_j_blocks):
            j0 = jb * 16
            col = 2 * j0 + 2 * iota
            jrow = j0 + iota
            for g in range(8):
                rows = (16 * g) % 64 + dvec
                cvec = 16 * g + dvec
                o = 0 if g < 4 else 1
                v = plsc.load_gather(slab, [rows, col + o])
                plsc.store_scatter(lbuf, [jrow, cvec], v * EMBED_SCALE)


def _sc_build_lines(wt, wt_tail):
    info = _info()
    nw = info.num_cores * info.num_subcores
    per_w = -(-NFULL // nw)

    mesh = plsc.VectorSubcoreMesh(core_axis_name="c", subcore_axis_name="s")

    @functools.partial(
        pl.kernel,
        mesh=mesh,
        out_type=jax.ShapeDtypeStruct((NLINES, 128), jnp.float32),
        scratch_types=[
            *[pltpu.VMEM((64, 128), jnp.float32) for _ in range(2)],
            *[pltpu.VMEM((64, 128), jnp.float32) for _ in range(2)],
            pltpu.VMEM((64, 128), jnp.float32),
            *[pltpu.SemaphoreType.DMA for _ in range(4)],
        ],
        compiler_params=pltpu.CompilerParams(needs_layout_passes=False),
    )
    def k(wt_hbm, tail_hbm, lines_hbm, sa0, sa1, lb0, lb1, tslab, g0, g1, s0, s1):
        slab = (sa0, sa1)
        lbuf = (lb0, lb1)
        gsem = (g0, g1)
        ssem = (s0, s1)
        wid = lax.axis_index("s") * info.num_cores + lax.axis_index("c")

        def fire_slab(b, blk):
            pltpu.async_copy(
                wt_hbm.at[:, pl.ds(blk * 128, 128)], slab[b], gsem[b]
            )

        for b in range(2):
            blk = wid + nw * b
            @pl.when(blk < NFULL)
            def _():
                fire_slab(b, blk)

        def body(i, carry):
            for b in range(2):
                blk = wid + nw * (i * 2 + b)

                @pl.when(blk < NFULL)
                def _():
                    pltpu.make_async_copy(
                        wt_hbm.at[:, pl.ds(0, 128)], slab[b], gsem[b]
                    ).wait()

                    @pl.when(i > 0)
                    def _():
                        pltpu.make_async_copy(
                            lbuf[b], lines_hbm.at[pl.ds(0, 64), :], ssem[b]
                        ).wait()

                    _transpose_slab(slab[b], lbuf[b], 4)
                    pltpu.async_copy(
                        lbuf[b], lines_hbm.at[pl.ds(blk * 64, 64), :], ssem[b]
                    )
                    nxt = blk + nw * 2
                    @pl.when(nxt < NFULL)
                    def _():
                        fire_slab(b, nxt)
            return carry

        lax.fori_loop(0, per_w // 2 + 1, body, 0)

        for b in range(2):
            blk = wid + nw * b
            @pl.when(blk < NFULL)
            def _():
                pltpu.make_async_copy(
                    lbuf[b], lines_hbm.at[pl.ds(0, 64), :], ssem[b]
                ).wait()

        @pl.when(wid == 0)
        def _():
            pltpu.sync_copy(tail_hbm, tslab)
            _transpose_slab(tslab, lbuf[0], 2)
            pltpu.sync_copy(
                lbuf[0].at[pl.ds(0, 32), :],
                lines_hbm.at[pl.ds(NFULL * 64, 32), :],
            )

    return k(wt, wt_tail)


def _sc_gather(idx_p, lines, S, N):
    info = _info()
    nw = info.num_cores * info.num_subcores
    C = 256
    n_chunks = (S * N) // C
    per_w = n_chunks // nw
    blocks_per_s = N // C
    assert per_w * nw == n_chunks and blocks_per_s * C == N
    blk_bits = blocks_per_s.bit_length() - 1

    mesh = plsc.VectorSubcoreMesh(core_axis_name="c", subcore_axis_name="s")

    @functools.partial(
        pl.kernel,
        mesh=mesh,
        out_type=jax.ShapeDtypeStruct((S, EMBED_DIM, N), jnp.float32),
        scratch_types=[
            *[pltpu.VMEM((C,), jnp.int32) for _ in range(2)],
            *[pltpu.VMEM((C,), jnp.int32) for _ in range(2)],
            *[pltpu.VMEM((C,), jnp.int32) for _ in range(2)],
            *[pltpu.VMEM((C, 128), jnp.float32) for _ in range(2)],
            *[pltpu.VMEM((EMBED_DIM, 128), jnp.float32) for _ in range(4)],
            *[pltpu.SemaphoreType.DMA for _ in range(4)],
        ],
        compiler_params=pltpu.CompilerParams(needs_layout_passes=False),
    )
    def k(idx_hbm, lines_hbm, out_hbm,
          p0_, p1_, l0_, l1_, h0_, h1_, lb0, lb1, t0, t1, t2, t3,
          g0, g1, s0, s1):
        pidx = (p0_, p1_)
        lidx = (l0_, l1_)
        h64 = (h0_, h1_)
        lbuf = (lb0, lb1)
        tbuf = ((t0, t1), (t2, t3))
        gsem = (g0, g1)
        ssem = (s0, s1)
        wid = lax.axis_index("s") * info.num_cores + lax.axis_index("c")
        base = wid * per_w
        iota = _iota16()

        def fire_gather(b, cid):
            pltpu.sync_copy(idx_hbm.at[pl.ds(cid * C, C)], pidx[b])

            @plsc.parallel_loop(0, C // 16, unroll=4)
            def _(r):
                t = pidx[b][pl.ds(r * 16, 16)]
                lidx[b][pl.ds(r * 16, 16)] = lax.shift_right_logical(t, 1)
                h64[b][pl.ds(r * 16, 16)] = lax.shift_left(t & 1, 6)

            pltpu.async_copy(lines_hbm.at[lidx[b]], lbuf[b], gsem[b])

        for b in range(2):
            fire_gather(b, base + b)

        def extract(b, nb):
            @plsc.parallel_loop(0, 16, unroll=1)
            def _(k2):
                dvec = (iota + k2) & 15
                for g in range(8):
                    r0 = nb * 128 + 16 * g
                    hv = h64[b][pl.ds(r0, 16)] + dvec
                    rows = r0 + iota
                    cols = 16 * g + iota
                    for db in range(4):
                        v = plsc.load_gather(lbuf[b], [rows, hv + db * 16])
                        plsc.store_scatter(
                            tbuf[b][nb], [db * 16 + dvec, cols], v
                        )

        def body(g, carry):
            for b in range(2):
                cid = base + g * 2 + b
                pltpu.make_async_copy(
                    lines_hbm.at[lidx[b]], lbuf[b], gsem[b]
                ).wait()

                @pl.when(g > 0)
                def _():
                    for nb in range(2):
                        pltpu.make_async_copy(
                            tbuf[b][nb], out_hbm.at[0, :, pl.ds(0, 128)],
                            ssem[b],
                        ).wait()

                s_pos = cid >> blk_bits
                n0 = (cid & (blocks_per_s - 1)) * C
                for nb in range(2):
                    extract(b, nb)
                    pltpu.async_copy(
                        tbuf[b][nb],
                        out_hbm.at[s_pos, :, pl.ds(n0 + nb * 128, 128)],
                        ssem[b],
                    )

                @pl.when(g + 1 < per_w // 2)
                def _():
                    fire_gather(b, cid + 2)
            return carry

        lax.fori_loop(0, per_w // 2, body, 0)

        for b in range(2):
            for nb in range(2):
                pltpu.make_async_copy(
                    tbuf[b][nb], out_hbm.at[0, :, pl.ds(0, 128)], ssem[b]
                ).wait()

    return k(idx_p, lines)


@jax.jit
def _run(tokens, weight):
    n, s = tokens.shape
    wt = weight.T
    wt_tail = lax.slice(wt, (0, NFULL * 128), (EMBED_DIM, VOCAB))
    wt_tail = jnp.pad(wt_tail, ((0, 0), (0, 128 - NRAG)))
    lines = _sc_build_lines(wt, wt_tail)
    idx_p = tokens.T.reshape(-1)
    out_phys = _sc_gather(idx_p, lines, s, n)
    return jnp.transpose(out_phys, (2, 0, 1))


def kernel(tokens, weight):
    return _run(tokens, weight)


# speedup vs baseline: 2.1139x; 1.0801x over previous
import functools
import math

import jax
import jax.numpy as jnp
from jax import lax
from jax.experimental import pallas as pl
from jax.experimental.pallas import tpu as pltpu
from jax.experimental.pallas import tpu_sc as plsc

EMBED_DIM = 64
EMBED_SCALE = math.sqrt(EMBED_DIM)
VOCAB = 1000000
NLINES = VOCAB // 2
NFULL = VOCAB // 128
NRAG = VOCAB - NFULL * 128

_INFO = None


def _info():
    global _INFO
    if _INFO is None:
        _INFO = plsc.get_sparse_core_info()
    return _INFO


def _iota16():
    return lax.iota(jnp.int32, 16)


def _transpose_slab(slab, lbuf, n_j_blocks):
    iota = _iota16()

    @plsc.parallel_loop(0, 16, unroll=2)
    def _(k):
        dvec = (iota + k) & 15
        for jb in range(n_j_blocks):
            j0 = jb * 16
            col = 2 * j0 + 2 * iota
            jrow = j0 + iota
            for g in range(8):
                rows = (16 * g) % 64 + dvec
                cvec = 16 * g + dvec
                o = 0 if g < 4 else 1
                v = plsc.load_gather(slab, [rows, col + o])
                plsc.store_scatter(lbuf, [jrow, cvec], v)


def _sc_build_lines(wt, wt_tail):
    info = _info()
    nw = info.num_cores * info.num_subcores
    per_w = -(-NFULL // nw)

    mesh = plsc.VectorSubcoreMesh(core_axis_name="c", subcore_axis_name="s")

    @functools.partial(
        pl.kernel,
        mesh=mesh,
        out_type=jax.ShapeDtypeStruct((NLINES, 128), jnp.float32),
        scratch_types=[
            *[pltpu.VMEM((64, 128), jnp.float32) for _ in range(2)],
            *[pltpu.VMEM((64, 128), jnp.float32) for _ in range(2)],
            pltpu.VMEM((64, 128), jnp.float32),
            *[pltpu.SemaphoreType.DMA for _ in range(4)],
        ],
        compiler_params=pltpu.CompilerParams(needs_layout_passes=False),
    )
    def k(wt_hbm, tail_hbm, lines_hbm, sa0, sa1, lb0, lb1, tslab, g0, g1, s0, s1):
        slab = (sa0, sa1)
        lbuf = (lb0, lb1)
        gsem = (g0, g1)
        ssem = (s0, s1)
        wid = lax.axis_index("s") * info.num_cores + lax.axis_index("c")

        def fire_slab(b, blk):
            pltpu.async_copy(
                wt_hbm.at[:, pl.ds(blk * 128, 128)], slab[b], gsem[b]
            )

        for b in range(2):
            blk = wid + nw * b
            @pl.when(blk < NFULL)
            def _():
                fire_slab(b, blk)

        def body(i, carry):
            for b in range(2):
                blk = wid + nw * (i * 2 + b)

                @pl.when(blk < NFULL)
                def _():
                    pltpu.make_async_copy(
                        wt_hbm.at[:, pl.ds(0, 128)], slab[b], gsem[b]
                    ).wait()

                    @pl.when(i > 0)
                    def _():
                        pltpu.make_async_copy(
                            lbuf[b], lines_hbm.at[pl.ds(0, 64), :], ssem[b]
                        ).wait()

                    _transpose_slab(slab[b], lbuf[b], 4)
                    pltpu.async_copy(
                        lbuf[b], lines_hbm.at[pl.ds(blk * 64, 64), :], ssem[b]
                    )
                    nxt = blk + nw * 2
                    @pl.when(nxt < NFULL)
                    def _():
                        fire_slab(b, nxt)
            return carry

        lax.fori_loop(0, per_w // 2 + 1, body, 0)

        for b in range(2):
            blk = wid + nw * b
            @pl.when(blk < NFULL)
            def _():
                pltpu.make_async_copy(
                    lbuf[b], lines_hbm.at[pl.ds(0, 64), :], ssem[b]
                ).wait()

        @pl.when(wid == 0)
        def _():
            pltpu.sync_copy(tail_hbm, tslab)
            _transpose_slab(tslab, lbuf[0], 2)
            pltpu.sync_copy(
                lbuf[0].at[pl.ds(0, 32), :],
                lines_hbm.at[pl.ds(NFULL * 64, 32), :],
            )

    return k(wt, wt_tail)


def _sc_gather(idx_p, lines, S, N):
    info = _info()
    nw = info.num_cores * info.num_subcores
    C = 256
    n_chunks = (S * N) // C
    per_w = n_chunks // nw
    blocks_per_s = N // C
    assert per_w * nw == n_chunks and blocks_per_s * C == N
    blk_bits = blocks_per_s.bit_length() - 1

    mesh = plsc.VectorSubcoreMesh(core_axis_name="c", subcore_axis_name="s")

    @functools.partial(
        pl.kernel,
        mesh=mesh,
        out_type=jax.ShapeDtypeStruct((S, EMBED_DIM, N), jnp.float32),
        scratch_types=[
            *[pltpu.VMEM((C,), jnp.int32) for _ in range(2)],
            *[pltpu.VMEM((C,), jnp.int32) for _ in range(2)],
            *[pltpu.VMEM((C,), jnp.int32) for _ in range(2)],
            *[pltpu.VMEM((C, 128), jnp.float32) for _ in range(2)],
            *[pltpu.VMEM((EMBED_DIM, 128), jnp.float32) for _ in range(4)],
            *[pltpu.SemaphoreType.DMA for _ in range(4)],
        ],
        compiler_params=pltpu.CompilerParams(needs_layout_passes=False),
    )
    def k(idx_hbm, lines_hbm, out_hbm,
          p0_, p1_, l0_, l1_, h0_, h1_, lb0, lb1, t0, t1, t2, t3,
          g0, g1, s0, s1):
        pidx = (p0_, p1_)
        lidx = (l0_, l1_)
        h64 = (h0_, h1_)
        lbuf = (lb0, lb1)
        tbuf = ((t0, t1), (t2, t3))
        gsem = (g0, g1)
        ssem = (s0, s1)
        wid = lax.axis_index("s") * info.num_cores + lax.axis_index("c")
        base = wid * per_w
        iota = _iota16()

        def fire_gather(b, cid):
            pltpu.sync_copy(idx_hbm.at[pl.ds(cid * C, C)], pidx[b])

            @plsc.parallel_loop(0, C // 16, unroll=4)
            def _(r):
                t = pidx[b][pl.ds(r * 16, 16)]
                lidx[b][pl.ds(r * 16, 16)] = lax.shift_right_logical(t, 1)
                h64[b][pl.ds(r * 16, 16)] = lax.shift_left(t & 1, 6)

            pltpu.async_copy(lines_hbm.at[lidx[b]], lbuf[b], gsem[b])

        for b in range(2):
            fire_gather(b, base + b)

        def extract(b, nb):
            @plsc.parallel_loop(0, 16, unroll=1)
            def _(k2):
                dvec = (iota + k2) & 15
                for g in range(8):
                    r0 = nb * 128 + 16 * g
                    hv = h64[b][pl.ds(r0, 16)] + dvec
                    rows = r0 + iota
                    cols = 16 * g + iota
                    for db in range(4):
                        v = plsc.load_gather(lbuf[b], [rows, hv + db * 16])
                        plsc.store_scatter(
                            tbuf[b][nb], [db * 16 + dvec, cols],
                            v * EMBED_SCALE,
                        )

        def body(g, carry):
            for b in range(2):
                cid = base + g * 2 + b
                pltpu.make_async_copy(
                    lines_hbm.at[lidx[b]], lbuf[b], gsem[b]
                ).wait()

                @pl.when(g > 0)
                def _():
                    for nb in range(2):
                        pltpu.make_async_copy(
                            tbuf[b][nb], out_hbm.at[0, :, pl.ds(0, 128)],
                            ssem[b],
                        ).wait()

                s_pos = cid >> blk_bits
                n0 = (cid & (blocks_per_s - 1)) * C
                for nb in range(2):
                    extract(b, nb)
                    pltpu.async_copy(
                        tbuf[b][nb],
                        out_hbm.at[s_pos, :, pl.ds(n0 + nb * 128, 128)],
                        ssem[b],
                    )

                @pl.when(g + 1 < per_w // 2)
                def _():
                    fire_gather(b, cid + 2)
            return carry

        lax.fori_loop(0, per_w // 2, body, 0)

        for b in range(2):
            for nb in range(2):
                pltpu.make_async_copy(
                    tbuf[b][nb], out_hbm.at[0, :, pl.ds(0, 128)], ssem[b]
                ).wait()

    return k(idx_p, lines)


@jax.jit
def _run(tokens, weight):
    n, s = tokens.shape
    wt = weight.T
    wt_tail = lax.slice(wt, (0, NFULL * 128), (EMBED_DIM, VOCAB))
    wt_tail = jnp.pad(wt_tail, ((0, 0), (0, 128 - NRAG)))
    lines = _sc_build_lines(wt, wt_tail)
    idx_p = tokens.T.reshape(-1)
    out_phys = _sc_gather(idx_p, lines, s, n)
    return jnp.transpose(out_phys, (2, 0, 1))


def kernel(tokens, weight):
    return _run(tokens, weight)
